# baseline (speedup 1.0000x reference)
import functools

import jax
import jax.numpy as jnp
from jax import lax
from jax.experimental import pallas as pl
from jax.experimental.pallas import tpu as pltpu
from jax.experimental.pallas import tpu_sc as plsc

N = 10000
E = 320000
F = 128
NC = 2
NS = 16
NW = NC * NS
CH = 128
KC = 80
EPT = CH * KC
EPAD = NW * EPT
NPAD = 10240
NP2 = 2 * NPAD
RPN = NPAD // NS
L = 16

_mesh = plsc.VectorSubcoreMesh(core_axis_name="c", subcore_axis_name="s",
                               num_cores=NC, num_subcores=NS)
_sc_params = pltpu.CompilerParams(use_tc_tiling_on_sc=False)


def _fill(ref, n, value):
    vec = jnp.full((L,), value, jnp.float32)
    for k in range(n // L):
        ref[pl.ds(k * L, L)] = vec


@functools.partial(
    pl.kernel,
    out_type=jax.ShapeDtypeStruct((NC * NP2,), jnp.float32),
    mesh=_mesh,
    compiler_params=_sc_params,
    scratch_types=[
        pltpu.VMEM((KC, CH), jnp.int32),
        pltpu.VMEM((KC, CH), jnp.int32),
        pltpu.VMEM((CH,), jnp.float32),
        pltpu.VMEM((RPN,), jnp.float32),
        pltpu.SemaphoreType.DMA,
        pltpu.VMEM_SHARED((NPAD,), jnp.float32),
        pltpu.VMEM_SHARED((NPAD,), jnp.float32),
    ],
)
def _deg_kernel(src0_hbm, dst0_hbm, degp_hbm,
                idx_s, idx_d, ones_v, zero_v, ssem, dsh_out, dsh_in):
    c = lax.axis_index("c")
    s = lax.axis_index("s")
    wid = c * NS + s
    _fill(ones_v, CH, 1.0)
    _fill(zero_v, RPN, 0.0)
    pltpu.sync_copy(zero_v, dsh_out.at[pl.ds(s * RPN, RPN)])
    pltpu.sync_copy(zero_v, dsh_in.at[pl.ds(s * RPN, RPN)])
    pltpu.sync_copy(src0_hbm.at[wid], idx_s)
    pltpu.sync_copy(dst0_hbm.at[wid], idx_d)
    plsc.subcore_barrier()

    def fire(j, carry):
        pltpu.async_copy(ones_v, dsh_out.at[idx_s.at[j]], ssem, add=True)
        pltpu.async_copy(ones_v, dsh_in.at[idx_d.at[j]], ssem, add=True)
        return carry

    lax.fori_loop(0, KC, fire, 0)

    def drain(j, carry):
        pltpu.make_async_copy(ones_v, dsh_out.at[idx_s.at[j]], ssem).wait()
        pltpu.make_async_copy(ones_v, dsh_in.at[idx_d.at[j]], ssem).wait()
        return carry

    lax.fori_loop(0, KC, drain, 0)
    plsc.subcore_barrier()
    pltpu.sync_copy(dsh_out.at[pl.ds(s * RPN, RPN)],
                    degp_hbm.at[pl.ds(c * NP2 + s * RPN, RPN)])
    pltpu.sync_copy(dsh_in.at[pl.ds(s * RPN, RPN)],
                    degp_hbm.at[pl.ds(c * NP2 + NPAD + s * RPN, RPN)])


@functools.partial(
    pl.kernel,
    out_type=jax.ShapeDtypeStruct((NC * NP2,), jnp.float32),
    mesh=_mesh,
    compiler_params=_sc_params,
    scratch_types=[
        pltpu.VMEM((KC, CH), jnp.int32),
        pltpu.VMEM((KC, CH), jnp.int32),
        pltpu.VMEM((KC, CH), jnp.float32),
        pltpu.VMEM((KC, CH), jnp.float32),
        pltpu.VMEM((RPN,), jnp.float32),
        pltpu.SemaphoreType.DMA,
        pltpu.SemaphoreType.DMA,
        pltpu.VMEM_SHARED((NPAD,), jnp.float32),
        pltpu.VMEM_SHARED((NPAD,), jnp.float32),
        pltpu.VMEM_SHARED((NPAD,), jnp.float32),
        pltpu.VMEM_SHARED((NPAD,), jnp.float32),
    ],
)
def _agg_kernel(src0_hbm, dst0_hbm, v_hbm, aggp_hbm,
                idx_s, idx_d, msg0, msg1, zero_v, gsem, ssem,
                ash0, ash1, vsh0, vsh1):
    c = lax.axis_index("c")
    s = lax.axis_index("s")
    wid = c * NS + s
    _fill(zero_v, RPN, 0.0)
    pltpu.sync_copy(zero_v, ash0.at[pl.ds(s * RPN, RPN)])
    pltpu.sync_copy(zero_v, ash1.at[pl.ds(s * RPN, RPN)])
    pltpu.sync_copy(v_hbm.at[pl.ds(s * RPN, RPN)],
                    vsh0.at[pl.ds(s * RPN, RPN)])
    pltpu.sync_copy(v_hbm.at[pl.ds(NPAD + s * RPN, RPN)],
                    vsh1.at[pl.ds(s * RPN, RPN)])
    pltpu.sync_copy(src0_hbm.at[wid], idx_s)
    pltpu.sync_copy(dst0_hbm.at[wid], idx_d)
    plsc.subcore_barrier()

    def gfire(j, carry):
        pltpu.async_copy(vsh0.at[idx_s.at[j]], msg0.at[j], gsem)
        pltpu.async_copy(vsh1.at[idx_s.at[j]], msg1.at[j], gsem)
        return carry

    lax.fori_loop(0, KC, gfire, 0)

    def pipe(j, carry):
        pltpu.make_async_copy(vsh0.at[idx_s.at[j]], msg0.at[j], gsem).wait()
        pltpu.make_async_copy(vsh1.at[idx_s.at[j]], msg1.at[j], gsem).wait()
        pltpu.async_copy(msg0.at[j], ash0.at[idx_d.at[j]], ssem, add=True)
        pltpu.async_copy(msg1.at[j], ash1.at[idx_d.at[j]], ssem, add=True)
        return carry

    lax.fori_loop(0, KC, pipe, 0)

    def sdrain(j, carry):
        pltpu.make_async_copy(msg0.at[j], ash0.at[idx_d.at[j]], ssem).wait()
        pltpu.make_async_copy(msg1.at[j], ash1.at[idx_d.at[j]], ssem).wait()
        return carry

    lax.fori_loop(0, KC, sdrain, 0)
    plsc.subcore_barrier()
    pltpu.sync_copy(ash0.at[pl.ds(s * RPN, RPN)],
                    aggp_hbm.at[pl.ds(c * NP2 + s * RPN, RPN)])
    pltpu.sync_copy(ash1.at[pl.ds(s * RPN, RPN)],
                    aggp_hbm.at[pl.ds(c * NP2 + NPAD + s * RPN, RPN)])


def _mm_body(x_ref, we_ref, fw_ref, wg_ref, cw_ref, u_ref):
    wc1 = jnp.dot(we_ref[...], fw_ref[...], preferred_element_type=jnp.float32)
    wc2 = jnp.dot(wg_ref[...], cw_ref[...], preferred_element_type=jnp.float32)
    u1 = jnp.dot(x_ref[...], wc1, preferred_element_type=jnp.float32)
    u2 = jnp.dot(x_ref[...], wc2, preferred_element_type=jnp.float32)
    pad = jnp.zeros((1, NPAD - N), jnp.float32)
    u_ref[...] = jnp.concatenate(
        [jnp.transpose(u1), pad, jnp.transpose(u2), pad], axis=1)


def _scale_body(u_ref, degp_ref, v_ref):
    deg4 = degp_ref[...].reshape(NC * 2, NPAD)
    dout = deg4[0:1, :] + deg4[2:3, :]
    ns = jnp.where(dout > 0, lax.rsqrt(jnp.maximum(dout, 1.0)), 0.0)
    v_ref[...] = u_ref[...] * jnp.concatenate([ns, ns], axis=1)


def _fin_body(aggp_ref, degp_ref, be_ref, fw_ref, fb_ref, bg_ref, cw_ref,
              cb_ref, s_ref, y_ref):
    deg4 = degp_ref[...].reshape(NC * 2, NPAD)
    agg4 = aggp_ref[...].reshape(NC * 2, NPAD)
    din = deg4[1:2, :] + deg4[3:4, :]
    nd = jnp.where(din > 0, lax.rsqrt(jnp.maximum(din, 1.0)), 0.0)
    agg0 = agg4[0:1, :] + agg4[2:3, :]
    agg1 = agg4[1:2, :] + agg4[3:4, :]
    c1 = jnp.sum(be_ref[...] * fw_ref[...]) + fb_ref[0, 0]
    c2 = jnp.sum(bg_ref[...] * cw_ref[...]) + cb_ref[0, 0]
    s_ref[...] = jnp.transpose(agg0 * nd + c1)[:N]
    y_ref[...] = jnp.transpose(agg1 * nd + c2)[:N]


_mm_call = pl.pallas_call(
    _mm_body,
    out_shape=jax.ShapeDtypeStruct((1, NP2), jnp.float32),
)

_scale_call = pl.pallas_call(
    _scale_body,
    out_shape=jax.ShapeDtypeStruct((1, NP2), jnp.float32),
)

_fin_call = pl.pallas_call(
    _fin_body,
    out_shape=[
        jax.ShapeDtypeStruct((N, 1), jnp.float32),
        jax.ShapeDtypeStruct((N, 1), jnp.float32),
    ],
)


def kernel(x, edge_index, W_est, b_est, fc_w, fc_b, W_gnn, b_gnn, cls_w, cls_b):
    src = edge_index[0]
    dst = edge_index[1]
    ppw = EPT - E // NW
    pad_blk = jnp.broadcast_to(
        N + (jnp.arange(ppw, dtype=jnp.int32) % (NPAD - N)), (NW, ppw))
    src0 = jnp.concatenate([src.reshape(NW, E // NW), pad_blk],
                           axis=1).reshape(NW, KC, CH)
    dst0 = jnp.concatenate([dst.reshape(NW, E // NW), pad_blk],
                           axis=1).reshape(NW, KC, CH)

    degp = _deg_kernel(src0, dst0)
    u01 = _mm_call(x, W_est, fc_w, W_gnn, cls_w)
    v01 = _scale_call(u01, degp)
    aggp = _agg_kernel(src0, dst0, v01.reshape(NP2))
    s_col, y_col = _fin_call(
        aggp, degp,
        b_est.reshape(1, F), fc_w.reshape(1, F), fc_b.reshape(1, 1),
        b_gnn.reshape(1, F), cls_w.reshape(1, F), cls_b.reshape(1, 1),
    )
    return (y_col, s_col)


# speedup vs baseline: 1.0892x; 1.0892x over previous
import functools

import jax
import jax.numpy as jnp
from jax import lax
from jax.experimental import pallas as pl
from jax.experimental.pallas import tpu as pltpu
from jax.experimental.pallas import tpu_sc as plsc

N = 10000
E = 320000
F = 128
NC = 2
NS = 16
NW = NC * NS
CH = 128
KC = 80
EPT = CH * KC
EPAD = NW * EPT
NPAD = 10240
NP2 = 2 * NPAD
RPN = NPAD // NS
L = 16

_mesh = plsc.VectorSubcoreMesh(core_axis_name="c", subcore_axis_name="s",
                               num_cores=NC, num_subcores=NS)
_sc_params = pltpu.CompilerParams(use_tc_tiling_on_sc=False)


def _fill(ref, n, value):
    vec = jnp.full((L,), value, jnp.float32)
    for k in range(n // L):
        ref[pl.ds(k * L, L)] = vec


@functools.partial(
    pl.kernel,
    out_type=jax.ShapeDtypeStruct((NC * NP2,), jnp.float32),
    mesh=_mesh,
    compiler_params=_sc_params,
    scratch_types=[
        pltpu.VMEM((KC, CH), jnp.int32),
        pltpu.VMEM((KC, CH), jnp.int32),
        pltpu.VMEM((CH,), jnp.float32),
        pltpu.VMEM((RPN,), jnp.float32),
        pltpu.SemaphoreType.DMA,
        pltpu.VMEM_SHARED((NPAD,), jnp.float32),
        pltpu.VMEM_SHARED((NPAD,), jnp.float32),
    ],
)
def _deg_kernel(src0_hbm, dst0_hbm, degp_hbm,
                idx_s, idx_d, ones_v, zero_v, ssem, dsh_out, dsh_in):
    c = lax.axis_index("c")
    s = lax.axis_index("s")
    wid = c * NS + s
    _fill(ones_v, CH, 1.0)
    _fill(zero_v, RPN, 0.0)
    pltpu.sync_copy(zero_v, dsh_out.at[pl.ds(s * RPN, RPN)])
    pltpu.sync_copy(zero_v, dsh_in.at[pl.ds(s * RPN, RPN)])
    pltpu.sync_copy(src0_hbm.at[wid], idx_s)
    pltpu.sync_copy(dst0_hbm.at[wid], idx_d)
    plsc.subcore_barrier()

    def fire(j, carry):
        pltpu.async_copy(ones_v, dsh_out.at[idx_s.at[j]], ssem, add=True)
        pltpu.async_copy(ones_v, dsh_in.at[idx_d.at[j]], ssem, add=True)
        return carry

    lax.fori_loop(0, KC, fire, 0)

    def drain(j, carry):
        pltpu.make_async_copy(ones_v, dsh_out.at[idx_s.at[j]], ssem).wait()
        pltpu.make_async_copy(ones_v, dsh_in.at[idx_d.at[j]], ssem).wait()
        return carry

    lax.fori_loop(0, KC, drain, 0)
    plsc.subcore_barrier()
    pltpu.sync_copy(dsh_out.at[pl.ds(s * RPN, RPN)],
                    degp_hbm.at[pl.ds(c * NP2 + s * RPN, RPN)])
    pltpu.sync_copy(dsh_in.at[pl.ds(s * RPN, RPN)],
                    degp_hbm.at[pl.ds(c * NP2 + NPAD + s * RPN, RPN)])


@functools.partial(
    pl.kernel,
    out_type=jax.ShapeDtypeStruct((NC * NP2,), jnp.float32),
    mesh=_mesh,
    compiler_params=_sc_params,
    scratch_types=[
        pltpu.VMEM((KC, CH), jnp.int32),
        pltpu.VMEM((KC, CH), jnp.int32),
        pltpu.VMEM((KC, CH), jnp.float32),
        pltpu.VMEM((KC, CH), jnp.float32),
        pltpu.VMEM((RPN,), jnp.float32),
        pltpu.SemaphoreType.DMA,
        pltpu.SemaphoreType.DMA,
        pltpu.VMEM_SHARED((NPAD,), jnp.float32),
        pltpu.VMEM_SHARED((NPAD,), jnp.float32),
        pltpu.VMEM_SHARED((NPAD,), jnp.float32),
        pltpu.VMEM_SHARED((NPAD,), jnp.float32),
    ],
)
def _agg_kernel(src0_hbm, dst0_hbm, v_hbm, aggp_hbm,
                idx_s, idx_d, msg0, msg1, zero_v, gsem, ssem,
                ash0, ash1, vsh0, vsh1):
    c = lax.axis_index("c")
    s = lax.axis_index("s")
    wid = c * NS + s
    _fill(zero_v, RPN, 0.0)
    pltpu.sync_copy(zero_v, ash0.at[pl.ds(s * RPN, RPN)])
    pltpu.sync_copy(zero_v, ash1.at[pl.ds(s * RPN, RPN)])
    pltpu.sync_copy(v_hbm.at[pl.ds(s * RPN, RPN)],
                    vsh0.at[pl.ds(s * RPN, RPN)])
    pltpu.sync_copy(v_hbm.at[pl.ds(NPAD + s * RPN, RPN)],
                    vsh1.at[pl.ds(s * RPN, RPN)])
    pltpu.sync_copy(src0_hbm.at[wid], idx_s)
    pltpu.sync_copy(dst0_hbm.at[wid], idx_d)
    plsc.subcore_barrier()

    def gfire(j, carry):
        pltpu.async_copy(vsh0.at[idx_s.at[j]], msg0.at[j], gsem)
        pltpu.async_copy(vsh1.at[idx_s.at[j]], msg1.at[j], gsem)
        return carry

    lax.fori_loop(0, KC, gfire, 0)

    def pipe(j, carry):
        pltpu.make_async_copy(vsh0.at[idx_s.at[j]], msg0.at[j], gsem).wait()
        pltpu.make_async_copy(vsh1.at[idx_s.at[j]], msg1.at[j], gsem).wait()
        pltpu.async_copy(msg0.at[j], ash0.at[idx_d.at[j]], ssem, add=True)
        pltpu.async_copy(msg1.at[j], ash1.at[idx_d.at[j]], ssem, add=True)
        return carry

    lax.fori_loop(0, KC, pipe, 0)

    def sdrain(j, carry):
        pltpu.make_async_copy(msg0.at[j], ash0.at[idx_d.at[j]], ssem).wait()
        pltpu.make_async_copy(msg1.at[j], ash1.at[idx_d.at[j]], ssem).wait()
        return carry

    lax.fori_loop(0, KC, sdrain, 0)
    plsc.subcore_barrier()
    pltpu.sync_copy(ash0.at[pl.ds(s * RPN, RPN)],
                    aggp_hbm.at[pl.ds(c * NP2 + s * RPN, RPN)])
    pltpu.sync_copy(ash1.at[pl.ds(s * RPN, RPN)],
                    aggp_hbm.at[pl.ds(c * NP2 + NPAD + s * RPN, RPN)])


def _mm_body(x_ref, we_ref, fw_ref, wg_ref, cw_ref, u_ref):
    wc1 = jnp.dot(we_ref[...], fw_ref[...], preferred_element_type=jnp.float32)
    wc2 = jnp.dot(wg_ref[...], cw_ref[...], preferred_element_type=jnp.float32)
    u1 = jnp.dot(x_ref[...], wc1, preferred_element_type=jnp.float32)
    u2 = jnp.dot(x_ref[...], wc2, preferred_element_type=jnp.float32)
    pad = jnp.zeros((1, NPAD - N), jnp.float32)
    u_ref[...] = jnp.concatenate(
        [jnp.transpose(u1), pad, jnp.transpose(u2), pad], axis=1)


def _scale_body(u_ref, deg4_ref, v_ref):
    dout = deg4_ref[0:1, :] + deg4_ref[2:3, :]
    ns = jnp.where(dout > 0, lax.rsqrt(jnp.maximum(dout, 1.0)), 0.0)
    v_ref[...] = u_ref[...] * jnp.concatenate([ns, ns], axis=1)


def _fin_body(agg4_ref, deg4_ref, be_ref, fw_ref, fb_ref, bg_ref, cw_ref,
              cb_ref, s_ref, y_ref):
    din = deg4_ref[1:2, :] + deg4_ref[3:4, :]
    nd = jnp.where(din > 0, lax.rsqrt(jnp.maximum(din, 1.0)), 0.0)
    agg0 = agg4_ref[0:1, :] + agg4_ref[2:3, :]
    agg1 = agg4_ref[1:2, :] + agg4_ref[3:4, :]
    c1 = jnp.sum(be_ref[...] * fw_ref[...]) + fb_ref[0, 0]
    c2 = jnp.sum(bg_ref[...] * cw_ref[...]) + cb_ref[0, 0]
    s_ref[...] = agg0 * nd + c1
    y_ref[...] = agg1 * nd + c2


_mm_call = pl.pallas_call(
    _mm_body,
    out_shape=jax.ShapeDtypeStruct((1, NP2), jnp.float32),
)

_scale_call = pl.pallas_call(
    _scale_body,
    out_shape=jax.ShapeDtypeStruct((1, NP2), jnp.float32),
)

_fin_call = pl.pallas_call(
    _fin_body,
    out_shape=[
        jax.ShapeDtypeStruct((1, NPAD), jnp.float32),
        jax.ShapeDtypeStruct((1, NPAD), jnp.float32),
    ],
)


def kernel(x, edge_index, W_est, b_est, fc_w, fc_b, W_gnn, b_gnn, cls_w, cls_b):
    src = edge_index[0]
    dst = edge_index[1]
    ppw = EPT - E // NW
    pad_blk = jnp.broadcast_to(
        N + (jnp.arange(ppw, dtype=jnp.int32) % (NPAD - N)), (NW, ppw))
    src0 = jnp.concatenate([src.reshape(NW, E // NW), pad_blk],
                           axis=1).reshape(NW, KC, CH)
    dst0 = jnp.concatenate([dst.reshape(NW, E // NW), pad_blk],
                           axis=1).reshape(NW, KC, CH)

    degp = _deg_kernel(src0, dst0)
    deg4 = degp.reshape(NC * 2, NPAD)
    u01 = _mm_call(x, W_est, fc_w, W_gnn, cls_w)
    v01 = _scale_call(u01, deg4)
    aggp = _agg_kernel(src0, dst0, v01.reshape(NP2))
    agg4 = aggp.reshape(NC * 2, NPAD)
    s_row, y_row = _fin_call(
        agg4, deg4,
        b_est.reshape(1, F), fc_w.reshape(1, F), fc_b.reshape(1, 1),
        b_gnn.reshape(1, F), cls_w.reshape(1, F), cls_b.reshape(1, 1),
    )
    return (y_row.reshape(NPAD, 1)[:N], s_row.reshape(NPAD, 1)[:N])
